# baseline (speedup 1.0000x reference)
import functools

import jax
import jax.numpy as jnp
from jax import lax
from jax.experimental import pallas as pl
from jax.experimental.pallas import tpu as pltpu
from jax.experimental.pallas import tpu_sc as plsc

_PC_RANGE = (-51.2, -51.2, -5.0, 51.2, 51.2, 3.0)
_EMBED = 256
_NCAMS = 6
_NLEV = 3
_LEVEL_HW = ((58, 100), (29, 50), (15, 25))

_NTEC = 32
_ROWS_PER_Q = _NCAMS * _NLEV * 4


def _build_table(feat0, feat1, feat2):
    tabs = []
    for feat in (feat0, feat1, feat2):
        B, N, C, H, W = feat.shape
        tabs.append(jnp.transpose(feat, (0, 1, 3, 4, 2)).reshape(B * N * H * W, C))
    return jnp.concatenate(tabs, axis=0)


def _build_indices(query, query_pos, reference_points, lidar2img, W_attn, b_attn,
                   img_h, img_w):
    B, Q, _ = query.shape
    attn = jax.nn.sigmoid((query + query_pos) @ W_attn.T + b_attn)
    attn = attn.reshape(B, Q, _NCAMS, _NLEV)

    pc = _PC_RANGE
    rp = jnp.stack([
        reference_points[..., 0] * (pc[3] - pc[0]) + pc[0],
        reference_points[..., 1] * (pc[4] - pc[1]) + pc[1],
        reference_points[..., 2] * (pc[5] - pc[2]) + pc[2],
        jnp.ones_like(reference_points[..., 0])], axis=-1)
    rp_cam = jnp.einsum('bnij,bqj->bnqi', lidar2img, rp)
    eps = 1e-5
    depth_ok = rp_cam[..., 2] > eps
    denom = jnp.maximum(rp_cam[..., 2], eps)
    gx = (rp_cam[..., 0] / denom / img_w - 0.5) * 2.0
    gy = (rp_cam[..., 1] / denom / img_h - 0.5) * 2.0
    mask = depth_ok & (gx > -1.0) & (gx < 1.0) & (gy > -1.0) & (gy < 1.0)
    mask_f = mask.astype(jnp.float32)

    bn = (jnp.arange(B * _NCAMS, dtype=jnp.int32)
          .reshape(B, _NCAMS, 1))
    idx_parts, wt_parts = [], []
    row_base = 0
    for lvl, (H, W) in enumerate(_LEVEL_HW):
        xi = (gx + 1.0) * W / 2.0 - 0.5
        yi = (gy + 1.0) * H / 2.0 - 0.5
        x0 = jnp.floor(xi)
        y0 = jnp.floor(yi)
        attn_l = jnp.transpose(attn[:, :, :, lvl], (0, 2, 1))
        for dx, dy in ((0, 0), (1, 0), (0, 1), (1, 1)):
            xc = x0 + dx
            yc = y0 + dy
            valid = ((xc >= 0) & (xc <= W - 1) & (yc >= 0) & (yc <= H - 1))
            wcorner = (1.0 - jnp.abs(xi - xc)) * (1.0 - jnp.abs(yi - yc))
            xcc = jnp.clip(xc, 0, W - 1).astype(jnp.int32)
            ycc = jnp.clip(yc, 0, H - 1).astype(jnp.int32)
            idx_parts.append(row_base + bn * (H * W) + ycc * W + xcc)
            wt_parts.append(wcorner * valid.astype(jnp.float32) * mask_f * attn_l)
        row_base += B * _NCAMS * H * W
    idx = jnp.stack(idx_parts, axis=0)
    wt = jnp.stack(wt_parts, axis=0)
    idx = jnp.transpose(idx, (1, 3, 2, 0)).reshape(B * Q, _ROWS_PER_Q)
    wt = jnp.transpose(wt, (1, 3, 2, 0)).reshape(B * Q, _ROWS_PER_Q)
    return idx, wt


_WPAD = 80
_QSLAB = 64


def _sc_gather_combine(table, idx, wt, qpad, qpt):
    mesh = plsc.VectorSubcoreMesh(core_axis_name="c", subcore_axis_name="s")

    @functools.partial(
        pl.kernel, mesh=mesh,
        out_type=jax.ShapeDtypeStruct((_NTEC, _QSLAB, 2, 128), jnp.float32),
        scratch_types=[
            pltpu.VMEM((_QSLAB, 128), jnp.int32),
            pltpu.VMEM((_QSLAB, 128), jnp.float32),
            pltpu.VMEM((_ROWS_PER_Q, 2, 128), jnp.float32),
            pltpu.VMEM((_ROWS_PER_Q, 2, 128), jnp.float32),
            pltpu.VMEM((_QSLAB, 2, 128), jnp.float32),
            pltpu.SemaphoreType.DMA,
            pltpu.SemaphoreType.DMA,
        ],
    )
    def k(table_hbm, idx_hbm, wt_hbm, out_hbm, idx_v, wt_v, rows_a, rows_b,
          out_v, sem_a, sem_b):
        c = lax.axis_index("c")
        s = lax.axis_index("s")
        wid = s * 2 + c
        pltpu.sync_copy(idx_hbm.at[wid], idx_v)
        pltpu.sync_copy(wt_hbm.at[wid], wt_v)

        def gather(qi, rows_v, sem):
            return pltpu.async_copy(
                table_hbm.at[idx_v.at[qi, pl.ds(0, _ROWS_PER_Q)]], rows_v, sem)

        def compute(qi, rows_v):
            def fma_block(accs, wvec, row0, njj):
                for jj in range(njj):
                    w = wvec[jj]
                    accs = tuple(
                        accs[sp * 8 + t]
                        + w * rows_v[row0 + jj, sp, pl.ds(t * 16, 16)]
                        for sp in range(2) for t in range(8))
                return accs

            def body_jb(jb, accs):
                wvec = wt_v[qi, pl.ds(jb * 16, 16)]
                return fma_block(accs, wvec, jb * 16, 16)

            accs = lax.fori_loop(
                0, 4, body_jb,
                tuple(jnp.zeros((16,), jnp.float32) for _ in range(16)))
            wvec = wt_v[qi, pl.ds(64, 16)]
            accs = fma_block(accs, wvec, 64, 8)
            for sp in range(2):
                for t in range(8):
                    out_v[qi, sp, pl.ds(t * 16, 16)] = accs[sp * 8 + t]

        gather(0, rows_a, sem_a).wait()

        def body_pair(t, carry):
            qa = 2 * t
            hb = gather(qa + 1, rows_b, sem_b)
            compute(qa, rows_a)
            ha = gather(qa + 2, rows_a, sem_a)
            hb.wait()
            compute(qa + 1, rows_b)
            ha.wait()
            return carry

        lax.fori_loop(0, (qpt - 1) // 2, body_pair, 0)
        compute(qpt - 1, rows_a)
        pltpu.sync_copy(out_v, out_hbm.at[wid])

    return k(table, idx, wt)


def kernel(query, query_pos, reference_points, feat0, feat1, feat2, lidar2img,
           W_attn, b_attn, W_out, b_out, W_pe1, b_pe1, W_pe2, b_pe2, img_h, img_w):
    B, Q, D = query.shape
    qpt = -(-(B * Q) // _NTEC)
    qpad = qpt * _NTEC
    assert qpt % 2 == 1 and qpt <= _QSLAB

    table = _build_table(feat0, feat1, feat2).reshape(-1, 2, 128)
    idx, wt = _build_indices(query, query_pos, reference_points, lidar2img,
                             W_attn, b_attn, img_h, img_w)
    pad = qpad - B * Q
    idx = jnp.concatenate([idx, jnp.zeros((pad, _ROWS_PER_Q), jnp.int32)], axis=0)
    wt = jnp.concatenate([wt, jnp.zeros((pad, _ROWS_PER_Q), jnp.float32)], axis=0)
    idx = jnp.pad(idx.reshape(_NTEC, qpt, _ROWS_PER_Q),
                  ((0, 0), (0, _QSLAB - qpt), (0, 128 - _ROWS_PER_Q)))
    wt = jnp.pad(wt.reshape(_NTEC, qpt, _ROWS_PER_Q),
                 ((0, 0), (0, _QSLAB - qpt), (0, 128 - _ROWS_PER_Q)))

    fused = _sc_gather_combine(table, idx, wt, qpad, qpt)
    fused = fused.reshape(_NTEC, _QSLAB, _EMBED)[:, :qpt]
    fused = fused.reshape(qpad, _EMBED)[:B * Q].reshape(B, Q, _EMBED)

    out = fused @ W_out.T + b_out

    x = jnp.clip(reference_points, 0.0, 1.0)
    x1 = jnp.clip(x, 1e-5, None)
    x2 = jnp.clip(1.0 - x, 1e-5, None)
    inv = jnp.log(x1 / x2)
    pos = jax.nn.relu(inv @ W_pe1.T + b_pe1) @ W_pe2.T + b_pe2
    return out + pos


# speedup vs baseline: 1.6106x; 1.6106x over previous
import functools

import jax
import jax.numpy as jnp
from jax import lax
from jax.experimental import pallas as pl
from jax.experimental.pallas import tpu as pltpu
from jax.experimental.pallas import tpu_sc as plsc

_PC_RANGE = (-51.2, -51.2, -5.0, 51.2, 51.2, 3.0)
_EMBED = 256
_NCAMS = 6
_NLEV = 3
_LEVEL_HW = ((58, 100), (29, 50), (15, 25))
_HWPAD = (6144, 1536, 512)

_NTEC = 32
_ROWS_PER_Q = _NCAMS * _NLEV * 4
_QSLAB = 64
_TBLK = 512


def _transpose_level(feat, tab_e, tab_o, row_base):
    BN, C, HW = feat.shape
    hwpad = -(-HW // _TBLK) * _TBLK
    nb = hwpad // _TBLK
    rtot = tab_e.shape[0]

    def body(x_ref, te_in, to_in, te_ref, to_ref):
        x = x_ref[0]
        te_ref[...] = jnp.swapaxes(x[0:128, :], 0, 1)
        to_ref[...] = jnp.swapaxes(x[128:256, :], 0, 1)

    blk0 = row_base // _TBLK
    out_spec = pl.BlockSpec((_TBLK, 128), lambda bn, hb: (blk0 + bn * nb + hb, 0))
    return pl.pallas_call(
        body,
        grid=(BN, nb),
        in_specs=[
            pl.BlockSpec((1, C, _TBLK), lambda bn, hb: (bn, 0, hb)),
            pl.BlockSpec(memory_space=pl.ANY),
            pl.BlockSpec(memory_space=pl.ANY),
        ],
        out_specs=[out_spec, out_spec],
        out_shape=[jax.ShapeDtypeStruct((rtot, 128), jnp.float32)] * 2,
        input_output_aliases={1: 0, 2: 1},
    )(feat, tab_e, tab_o)


def _build_tables(feat0, feat1, feat2):
    B, N = feat0.shape[:2]
    rtot = B * N * sum(_HWPAD)
    tab_e = tab_o = None
    row_base = 0
    for lvl, feat in enumerate((feat0, feat1, feat2)):
        Bf, Nf, C, H, W = feat.shape
        f = feat.reshape(Bf * Nf, C, H * W)
        if tab_e is None:
            BN, _, HW = f.shape
            nb = (-(-HW // _TBLK) * _TBLK) // _TBLK

            def body(x_ref, te_ref, to_ref):
                x = x_ref[0]
                te_ref[...] = jnp.swapaxes(x[0:128, :], 0, 1)
                to_ref[...] = jnp.swapaxes(x[128:256, :], 0, 1)

            out_spec = pl.BlockSpec((_TBLK, 128), lambda bn, hb: (bn * nb + hb, 0))
            tab_e, tab_o = pl.pallas_call(
                body,
                grid=(BN, nb),
                in_specs=[pl.BlockSpec((1, C, _TBLK), lambda bn, hb: (bn, 0, hb))],
                out_specs=[out_spec, out_spec],
                out_shape=[jax.ShapeDtypeStruct((rtot, 128), jnp.float32)] * 2,
            )(f)
        else:
            tab_e, tab_o = _transpose_level(f, tab_e, tab_o, row_base)
        row_base += B * N * _HWPAD[lvl]
    return tab_e, tab_o


def _build_indices(query, query_pos, reference_points, lidar2img, W_attn, b_attn,
                   img_h, img_w):
    B, Q, _ = query.shape
    attn = jax.nn.sigmoid((query + query_pos) @ W_attn.T + b_attn)
    attn = attn.reshape(B, Q, _NCAMS, _NLEV)

    pc = _PC_RANGE
    rp = jnp.stack([
        reference_points[..., 0] * (pc[3] - pc[0]) + pc[0],
        reference_points[..., 1] * (pc[4] - pc[1]) + pc[1],
        reference_points[..., 2] * (pc[5] - pc[2]) + pc[2],
        jnp.ones_like(reference_points[..., 0])], axis=-1)
    rp_cam = jnp.einsum('bnij,bqj->bnqi', lidar2img, rp)
    eps = 1e-5
    depth_ok = rp_cam[..., 2] > eps
    denom = jnp.maximum(rp_cam[..., 2], eps)
    gx = (rp_cam[..., 0] / denom / img_w - 0.5) * 2.0
    gy = (rp_cam[..., 1] / denom / img_h - 0.5) * 2.0
    mask = depth_ok & (gx > -1.0) & (gx < 1.0) & (gy > -1.0) & (gy < 1.0)
    mask_f = mask.astype(jnp.float32)

    bn = (jnp.arange(B * _NCAMS, dtype=jnp.int32)
          .reshape(B, _NCAMS, 1))
    idx_parts, wt_parts = [], []
    row_base = 0
    for lvl, (H, W) in enumerate(_LEVEL_HW):
        xi = (gx + 1.0) * W / 2.0 - 0.5
        yi = (gy + 1.0) * H / 2.0 - 0.5
        x0 = jnp.floor(xi)
        y0 = jnp.floor(yi)
        attn_l = jnp.transpose(attn[:, :, :, lvl], (0, 2, 1))
        for dx, dy in ((0, 0), (1, 0), (0, 1), (1, 1)):
            xc = x0 + dx
            yc = y0 + dy
            valid = ((xc >= 0) & (xc <= W - 1) & (yc >= 0) & (yc <= H - 1))
            wcorner = (1.0 - jnp.abs(xi - xc)) * (1.0 - jnp.abs(yi - yc))
            xcc = jnp.clip(xc, 0, W - 1).astype(jnp.int32)
            ycc = jnp.clip(yc, 0, H - 1).astype(jnp.int32)
            idx_parts.append(row_base + bn * _HWPAD[lvl] + ycc * W + xcc)
            wt_parts.append(wcorner * valid.astype(jnp.float32) * mask_f * attn_l)
        row_base += B * _NCAMS * _HWPAD[lvl]
    idx = jnp.stack(idx_parts, axis=0)
    wt = jnp.stack(wt_parts, axis=0)
    idx = jnp.transpose(idx, (1, 3, 2, 0)).reshape(B * Q, _ROWS_PER_Q)
    wt = jnp.transpose(wt, (1, 3, 2, 0)).reshape(B * Q, _ROWS_PER_Q)
    return idx, wt


def _sc_gather_combine(tab_e, tab_o, idx, wt, qpt):
    mesh = plsc.VectorSubcoreMesh(core_axis_name="c", subcore_axis_name="s")
    rows_t = pltpu.VMEM((_ROWS_PER_Q, 128), jnp.float32)

    @functools.partial(
        pl.kernel, mesh=mesh,
        out_type=jax.ShapeDtypeStruct((_NTEC, _QSLAB, 2, 128), jnp.float32),
        scratch_types=[
            pltpu.VMEM((_QSLAB, 128), jnp.int32),
            pltpu.VMEM((_QSLAB, 128), jnp.float32),
            rows_t, rows_t, rows_t, rows_t,
            pltpu.VMEM((_QSLAB, 2, 128), jnp.float32),
            pltpu.SemaphoreType.DMA,
            pltpu.SemaphoreType.DMA,
            pltpu.SemaphoreType.DMA,
            pltpu.SemaphoreType.DMA,
        ],
    )
    def k(te_hbm, to_hbm, idx_hbm, wt_hbm, out_hbm, idx_v, wt_v,
          rows_ae, rows_ao, rows_be, rows_bo, out_v,
          sem_ae, sem_ao, sem_be, sem_bo):
        c = lax.axis_index("c")
        s = lax.axis_index("s")
        wid = s * 2 + c
        pltpu.sync_copy(idx_hbm.at[wid], idx_v)
        pltpu.sync_copy(wt_hbm.at[wid], wt_v)

        def gather(qi, rows_e, rows_o, sem_e, sem_o):
            isl = idx_v.at[qi, pl.ds(0, _ROWS_PER_Q)]
            he = pltpu.async_copy(te_hbm.at[isl], rows_e, sem_e)
            ho = pltpu.async_copy(to_hbm.at[isl], rows_o, sem_o)
            return he, ho

        def compute(qi, rows_e, rows_o):
            def fma_block(accs, wvec, row0, njj):
                for jj in range(njj):
                    w = wvec[jj]
                    accs = tuple(
                        accs[sp * 8 + t]
                        + w * (rows_e, rows_o)[sp][row0 + jj, pl.ds(t * 16, 16)]
                        for sp in range(2) for t in range(8))
                return accs

            def body_jb(jb, accs):
                wvec = wt_v[qi, pl.ds(jb * 16, 16)]
                return fma_block(accs, wvec, jb * 16, 16)

            accs = lax.fori_loop(
                0, 4, body_jb,
                tuple(jnp.zeros((16,), jnp.float32) for _ in range(16)))
            wvec = wt_v[qi, pl.ds(64, 16)]
            accs = fma_block(accs, wvec, 64, 8)
            for sp in range(2):
                for t in range(8):
                    out_v[qi, sp, pl.ds(t * 16, 16)] = accs[sp * 8 + t]

        ha = gather(0, rows_ae, rows_ao, sem_ae, sem_ao)
        ha[0].wait()
        ha[1].wait()

        def body_pair(t, carry):
            qa = 2 * t
            hb = gather(qa + 1, rows_be, rows_bo, sem_be, sem_bo)
            compute(qa, rows_ae, rows_ao)
            ha = gather(qa + 2, rows_ae, rows_ao, sem_ae, sem_ao)
            hb[0].wait()
            hb[1].wait()
            compute(qa + 1, rows_be, rows_bo)
            ha[0].wait()
            ha[1].wait()
            return carry

        lax.fori_loop(0, (qpt - 1) // 2, body_pair, 0)
        compute(qpt - 1, rows_ae, rows_ao)
        pltpu.sync_copy(out_v, out_hbm.at[wid])

    return k(tab_e, tab_o, idx, wt)


def kernel(query, query_pos, reference_points, feat0, feat1, feat2, lidar2img,
           W_attn, b_attn, W_out, b_out, W_pe1, b_pe1, W_pe2, b_pe2, img_h, img_w):
    B, Q, D = query.shape
    qpt = -(-(B * Q) // _NTEC)
    qpad = qpt * _NTEC
    assert qpt % 2 == 1 and qpt <= _QSLAB

    tab_e, tab_o = _build_tables(feat0, feat1, feat2)
    idx, wt = _build_indices(query, query_pos, reference_points, lidar2img,
                             W_attn, b_attn, img_h, img_w)
    pad = qpad - B * Q
    idx = jnp.concatenate([idx, jnp.zeros((pad, _ROWS_PER_Q), jnp.int32)], axis=0)
    wt = jnp.concatenate([wt, jnp.zeros((pad, _ROWS_PER_Q), jnp.float32)], axis=0)
    idx = jnp.pad(idx.reshape(_NTEC, qpt, _ROWS_PER_Q),
                  ((0, 0), (0, _QSLAB - qpt), (0, 128 - _ROWS_PER_Q)))
    wt = jnp.pad(wt.reshape(_NTEC, qpt, _ROWS_PER_Q),
                 ((0, 0), (0, _QSLAB - qpt), (0, 128 - _ROWS_PER_Q)))

    fused = _sc_gather_combine(tab_e, tab_o, idx, wt, qpt)
    fused = fused.reshape(_NTEC, _QSLAB, _EMBED)[:, :qpt]
    fused = fused.reshape(qpad, _EMBED)[:B * Q].reshape(B, Q, _EMBED)

    out = fused @ W_out.T + b_out

    x = jnp.clip(reference_points, 0.0, 1.0)
    x1 = jnp.clip(x, 1e-5, None)
    x2 = jnp.clip(1.0 - x, 1e-5, None)
    inv = jnp.log(x1 / x2)
    pos = jax.nn.relu(inv @ W_pe1.T + b_pe1) @ W_pe2.T + b_pe2
    return out + pos


# speedup vs baseline: 1.6140x; 1.0021x over previous
import functools

import jax
import jax.numpy as jnp
from jax import lax
from jax.experimental import pallas as pl
from jax.experimental.pallas import tpu as pltpu
from jax.experimental.pallas import tpu_sc as plsc

_PC_RANGE = (-51.2, -51.2, -5.0, 51.2, 51.2, 3.0)
_EMBED = 256
_NCAMS = 6
_NLEV = 3
_LEVEL_HW = ((58, 100), (29, 50), (15, 25))
_HWPAD = (6144, 1536, 512)

_NTEC = 32
_ROWS_PER_Q = _NCAMS * _NLEV * 4
_QSLAB = 64
_TBLK = 512


def _transpose_level(feat, tab_e, tab_o, row_base):
    BN, C, HW = feat.shape
    hwpad = -(-HW // _TBLK) * _TBLK
    nb = hwpad // _TBLK
    rtot = tab_e.shape[0]

    def body(x_ref, te_in, to_in, te_ref, to_ref):
        x = x_ref[0]
        te_ref[...] = jnp.swapaxes(x[0:128, :], 0, 1)
        to_ref[...] = jnp.swapaxes(x[128:256, :], 0, 1)

    blk0 = row_base // _TBLK
    out_spec = pl.BlockSpec((_TBLK, 128), lambda bn, hb: (blk0 + bn * nb + hb, 0))
    return pl.pallas_call(
        body,
        grid=(BN, nb),
        in_specs=[
            pl.BlockSpec((1, C, _TBLK), lambda bn, hb: (bn, 0, hb)),
            pl.BlockSpec(memory_space=pl.ANY),
            pl.BlockSpec(memory_space=pl.ANY),
        ],
        out_specs=[out_spec, out_spec],
        out_shape=[jax.ShapeDtypeStruct((rtot, 128), jnp.float32)] * 2,
        input_output_aliases={1: 0, 2: 1},
    )(feat, tab_e, tab_o)


def _build_tables(feat0, feat1, feat2):
    B, N = feat0.shape[:2]
    rtot = B * N * sum(_HWPAD)
    tab_e = tab_o = None
    row_base = 0
    for lvl, feat in enumerate((feat0, feat1, feat2)):
        Bf, Nf, C, H, W = feat.shape
        f = feat.reshape(Bf * Nf, C, H * W)
        if tab_e is None:
            BN, _, HW = f.shape
            nb = (-(-HW // _TBLK) * _TBLK) // _TBLK

            def body(x_ref, te_ref, to_ref):
                x = x_ref[0]
                te_ref[...] = jnp.swapaxes(x[0:128, :], 0, 1)
                to_ref[...] = jnp.swapaxes(x[128:256, :], 0, 1)

            out_spec = pl.BlockSpec((_TBLK, 128), lambda bn, hb: (bn * nb + hb, 0))
            tab_e, tab_o = pl.pallas_call(
                body,
                grid=(BN, nb),
                in_specs=[pl.BlockSpec((1, C, _TBLK), lambda bn, hb: (bn, 0, hb))],
                out_specs=[out_spec, out_spec],
                out_shape=[jax.ShapeDtypeStruct((rtot, 128), jnp.float32)] * 2,
            )(f)
        else:
            tab_e, tab_o = _transpose_level(f, tab_e, tab_o, row_base)
        row_base += B * N * _HWPAD[lvl]
    return tab_e, tab_o


def _build_indices(query, query_pos, reference_points, lidar2img, W_attn, b_attn,
                   img_h, img_w):
    B, Q, _ = query.shape
    attn = jax.nn.sigmoid((query + query_pos) @ W_attn.T + b_attn)
    attn = attn.reshape(B, Q, _NCAMS, _NLEV)

    pc = _PC_RANGE
    rp = jnp.stack([
        reference_points[..., 0] * (pc[3] - pc[0]) + pc[0],
        reference_points[..., 1] * (pc[4] - pc[1]) + pc[1],
        reference_points[..., 2] * (pc[5] - pc[2]) + pc[2],
        jnp.ones_like(reference_points[..., 0])], axis=-1)
    rp_cam = jnp.einsum('bnij,bqj->bnqi', lidar2img, rp)
    eps = 1e-5
    depth_ok = rp_cam[..., 2] > eps
    denom = jnp.maximum(rp_cam[..., 2], eps)
    gx = (rp_cam[..., 0] / denom / img_w - 0.5) * 2.0
    gy = (rp_cam[..., 1] / denom / img_h - 0.5) * 2.0
    mask = depth_ok & (gx > -1.0) & (gx < 1.0) & (gy > -1.0) & (gy < 1.0)
    mask_f = mask.astype(jnp.float32)

    bn = (jnp.arange(B * _NCAMS, dtype=jnp.int32)
          .reshape(B, _NCAMS, 1))
    idx_parts, wt_parts = [], []
    row_base = 0
    for lvl, (H, W) in enumerate(_LEVEL_HW):
        xi = (gx + 1.0) * W / 2.0 - 0.5
        yi = (gy + 1.0) * H / 2.0 - 0.5
        x0 = jnp.floor(xi)
        y0 = jnp.floor(yi)
        attn_l = jnp.transpose(attn[:, :, :, lvl], (0, 2, 1))
        for dx, dy in ((0, 0), (1, 0), (0, 1), (1, 1)):
            xc = x0 + dx
            yc = y0 + dy
            valid = ((xc >= 0) & (xc <= W - 1) & (yc >= 0) & (yc <= H - 1))
            wcorner = (1.0 - jnp.abs(xi - xc)) * (1.0 - jnp.abs(yi - yc))
            xcc = jnp.clip(xc, 0, W - 1).astype(jnp.int32)
            ycc = jnp.clip(yc, 0, H - 1).astype(jnp.int32)
            idx_parts.append(row_base + bn * _HWPAD[lvl] + ycc * W + xcc)
            wt_parts.append(wcorner * valid.astype(jnp.float32) * mask_f * attn_l)
        row_base += B * _NCAMS * _HWPAD[lvl]
    idx = jnp.stack(idx_parts, axis=0)
    wt = jnp.stack(wt_parts, axis=0)
    idx = jnp.transpose(idx, (1, 3, 2, 0)).reshape(B * Q, _ROWS_PER_Q)
    wt = jnp.transpose(wt, (1, 3, 2, 0)).reshape(B * Q, _ROWS_PER_Q)
    return idx, wt


def _sc_gather_combine(tab_e, tab_o, idx, wt, qpt):
    mesh = plsc.VectorSubcoreMesh(core_axis_name="c", subcore_axis_name="s")
    rows_t = pltpu.VMEM((_ROWS_PER_Q, 128), jnp.float32)

    @functools.partial(
        pl.kernel, mesh=mesh,
        compiler_params=pltpu.CompilerParams(use_tc_tiling_on_sc=True),
        out_type=jax.ShapeDtypeStruct((_NTEC, _QSLAB, 2, 128), jnp.float32),
        scratch_types=[
            pltpu.VMEM((_QSLAB, 128), jnp.int32),
            pltpu.VMEM((_QSLAB, 128), jnp.float32),
            rows_t, rows_t, rows_t, rows_t,
            pltpu.VMEM((_QSLAB, 2, 128), jnp.float32),
            pltpu.SemaphoreType.DMA,
            pltpu.SemaphoreType.DMA,
            pltpu.SemaphoreType.DMA,
            pltpu.SemaphoreType.DMA,
        ],
    )
    def k(te_hbm, to_hbm, idx_hbm, wt_hbm, out_hbm, idx_v, wt_v,
          rows_ae, rows_ao, rows_be, rows_bo, out_v,
          sem_ae, sem_ao, sem_be, sem_bo):
        c = lax.axis_index("c")
        s = lax.axis_index("s")
        wid = s * 2 + c
        pltpu.sync_copy(idx_hbm.at[wid], idx_v)
        pltpu.sync_copy(wt_hbm.at[wid], wt_v)

        def gather(qi, rows_e, rows_o, sem_e, sem_o):
            isl = idx_v.at[qi, pl.ds(0, _ROWS_PER_Q)]
            he = pltpu.async_copy(te_hbm.at[isl], rows_e, sem_e)
            ho = pltpu.async_copy(to_hbm.at[isl], rows_o, sem_o)
            return he, ho

        def compute(qi, rows_e, rows_o):
            def fma_block(accs, wvec, row0, njj):
                for jj in range(njj):
                    w = wvec[jj]
                    accs = tuple(
                        accs[sp * 8 + t]
                        + w * (rows_e, rows_o)[sp][row0 + jj, pl.ds(t * 16, 16)]
                        for sp in range(2) for t in range(8))
                return accs

            def body_jb(jb, accs):
                wvec = wt_v[qi, pl.ds(jb * 16, 16)]
                return fma_block(accs, wvec, jb * 16, 16)

            accs = lax.fori_loop(
                0, 4, body_jb,
                tuple(jnp.zeros((16,), jnp.float32) for _ in range(16)))
            wvec = wt_v[qi, pl.ds(64, 16)]
            accs = fma_block(accs, wvec, 64, 8)
            for sp in range(2):
                for t in range(8):
                    out_v[qi, sp, pl.ds(t * 16, 16)] = accs[sp * 8 + t]

        ha = gather(0, rows_ae, rows_ao, sem_ae, sem_ao)
        ha[0].wait()
        ha[1].wait()

        def body_pair(t, carry):
            qa = 2 * t
            hb = gather(qa + 1, rows_be, rows_bo, sem_be, sem_bo)
            compute(qa, rows_ae, rows_ao)
            ha = gather(qa + 2, rows_ae, rows_ao, sem_ae, sem_ao)
            hb[0].wait()
            hb[1].wait()
            compute(qa + 1, rows_be, rows_bo)
            ha[0].wait()
            ha[1].wait()
            return carry

        lax.fori_loop(0, (qpt - 1) // 2, body_pair, 0)
        compute(qpt - 1, rows_ae, rows_ao)
        pltpu.sync_copy(out_v, out_hbm.at[wid])

    return k(tab_e, tab_o, idx, wt)


def kernel(query, query_pos, reference_points, feat0, feat1, feat2, lidar2img,
           W_attn, b_attn, W_out, b_out, W_pe1, b_pe1, W_pe2, b_pe2, img_h, img_w):
    B, Q, D = query.shape
    qpt = -(-(B * Q) // _NTEC)
    qpad = qpt * _NTEC
    assert qpt % 2 == 1 and qpt <= _QSLAB

    tab_e, tab_o = _build_tables(feat0, feat1, feat2)
    idx, wt = _build_indices(query, query_pos, reference_points, lidar2img,
                             W_attn, b_attn, img_h, img_w)
    pad = qpad - B * Q
    idx = jnp.concatenate([idx, jnp.zeros((pad, _ROWS_PER_Q), jnp.int32)], axis=0)
    wt = jnp.concatenate([wt, jnp.zeros((pad, _ROWS_PER_Q), jnp.float32)], axis=0)
    idx = jnp.pad(idx.reshape(_NTEC, qpt, _ROWS_PER_Q),
                  ((0, 0), (0, _QSLAB - qpt), (0, 128 - _ROWS_PER_Q)))
    wt = jnp.pad(wt.reshape(_NTEC, qpt, _ROWS_PER_Q),
                 ((0, 0), (0, _QSLAB - qpt), (0, 128 - _ROWS_PER_Q)))

    fused = _sc_gather_combine(tab_e, tab_o, idx, wt, qpt)
    fused = fused.reshape(_NTEC, _QSLAB, _EMBED)[:, :qpt]
    fused = fused.reshape(qpad, _EMBED)[:B * Q].reshape(B, Q, _EMBED)

    out = fused @ W_out.T + b_out

    x = jnp.clip(reference_points, 0.0, 1.0)
    x1 = jnp.clip(x, 1e-5, None)
    x2 = jnp.clip(1.0 - x, 1e-5, None)
    inv = jnp.log(x1 / x2)
    pos = jax.nn.relu(inv @ W_pe1.T + b_pe1) @ W_pe2.T + b_pe2
    return out + pos
